# Initial kernel scaffold; baseline (speedup 1.0000x reference)
import jax
import jax.numpy as jnp
from jax.experimental import pallas as pl


def kernel(x, edge_index, W, bias, gamma, beta):
    raise NotImplementedError("write your pallas kernel here")


# speedup vs baseline: 10.4913x; 10.4913x over previous
import functools

import jax
import jax.numpy as jnp
from jax import lax
from jax.experimental import pallas as pl
from jax.experimental.pallas import tpu as pltpu
from jax.experimental.pallas import tpu_sc as plsc

N = 10000
D = 128
E = 320000

NP = 10240
EP = 327680
NC = 2
NS = 16
NT = NC * NS
EPT = EP // NT
CK = 128
NCHUNK = EPT // CK
RPT = NP // NS
R = 256
G = NP // R

_mesh = plsc.VectorSubcoreMesh(core_axis_name="c", subcore_axis_name="s")


@functools.partial(
    pl.kernel,
    mesh=_mesh,
    out_type=jax.ShapeDtypeStruct((NT, NP), jnp.float32),
    scratch_types=[
        pltpu.VMEM((EPT,), jnp.int32),
        pltpu.VMEM((NP,), jnp.float32),
    ],
    compiler_params=pltpu.CompilerParams(needs_layout_passes=False),
)
def _p1_hist(dst_hbm, counts_hbm, dst_v, hist):
    c = lax.axis_index("c")
    s = lax.axis_index("s")
    wid = c * NS + s

    def zbody(i, carry):
        hist[pl.ds(i * 16, 16)] = jnp.zeros((16,), jnp.float32)
        return carry

    lax.fori_loop(0, NP // 16, zbody, 0)

    pltpu.sync_copy(dst_hbm.at[pl.ds(wid * EPT, EPT)], dst_v)
    ones = jnp.full((16,), 1.0, jnp.float32)

    def body(g, carry):
        idx = dst_v[pl.ds(g * 16, 16)]
        plsc.addupdate_scatter(hist, [idx], ones)
        return carry

    lax.fori_loop(0, EPT // 16, body, 0)
    pltpu.sync_copy(hist, counts_hbm.at[wid])


@functools.partial(
    pl.kernel,
    mesh=_mesh,
    out_type=jax.ShapeDtypeStruct((NC, NP, D), jnp.float32),
    scratch_types=[
        pltpu.VMEM_SHARED((NP, D), jnp.float32),
        pltpu.VMEM((CK,), jnp.int32),
        pltpu.VMEM((CK,), jnp.int32),
        pltpu.VMEM((CK, D), jnp.float32),
        pltpu.SemaphoreType.DMA,
    ],
    compiler_params=pltpu.CompilerParams(needs_layout_passes=False),
)
def _p3_agg(hs_hbm, src_hbm, dst_hbm, zeros_hbm, out_hbm, agg, idx_s, idx_d, rows, sem):
    c = lax.axis_index("c")
    s = lax.axis_index("s")
    pltpu.sync_copy(zeros_hbm.at[pl.ds(s * RPT, RPT)], agg.at[pl.ds(s * RPT, RPT)])
    plsc.subcore_barrier()

    base0 = c * (EP // NC) + s * EPT

    def body(ch, carry):
        b = base0 + ch * CK
        pltpu.sync_copy(src_hbm.at[pl.ds(b, CK)], idx_s)
        pltpu.async_copy(hs_hbm.at[idx_s], rows, sem).wait()
        pltpu.sync_copy(dst_hbm.at[pl.ds(b, CK)], idx_d)
        pltpu.sync_copy(rows, agg.at[idx_d], add=True)
        return carry

    lax.fori_loop(0, NCHUNK, body, 0)
    plsc.subcore_barrier()
    pltpu.sync_copy(agg.at[pl.ds(s * RPT, RPT)], out_hbm.at[c, pl.ds(s * RPT, RPT)])


def _p2_body(x_ref, w_ref, cnt_ref, hs_ref):
    h = jnp.dot(x_ref[...], w_ref[...], preferred_element_type=jnp.float32)
    deg = jnp.sum(cnt_ref[...], axis=0) + 1.0
    dinv = lax.rsqrt(deg)
    hs_ref[...] = h * dinv[:, None]


def _p2(x_pad, W, counts):
    return pl.pallas_call(
        _p2_body,
        grid=(G,),
        in_specs=[
            pl.BlockSpec((R, D), lambda i: (i, 0)),
            pl.BlockSpec((D, D), lambda i: (0, 0)),
            pl.BlockSpec((NT, R), lambda i: (0, i)),
        ],
        out_specs=pl.BlockSpec((R, D), lambda i: (i, 0)),
        out_shape=jax.ShapeDtypeStruct((NP, D), jnp.float32),
    )(x_pad, W, counts)


def _p4a_body(agg_ref, hs_ref, cnt_ref, bias_ref, pre_ref, st_ref):
    i = pl.program_id(0)
    deg = jnp.sum(cnt_ref[...], axis=0) + 1.0
    dinv = lax.rsqrt(deg)
    tot = agg_ref[0] + agg_ref[1] + hs_ref[...]
    pre = tot * dinv[:, None] + bias_ref[...]
    pre_ref[...] = pre

    rows = i * R + lax.broadcasted_iota(jnp.int32, (R, 1), 0)
    m = (rows < N).astype(jnp.float32)
    pm = pre * m
    s1 = jnp.sum(pm, axis=0)
    s2 = jnp.sum(pm * pm, axis=0)
    prev = jnp.where(i == 0, 0.0, st_ref[...])
    s1n = prev[0] + s1
    s2n = prev[1] + s2
    st_ref[0:1, :] = s1n[None, :]
    st_ref[1:2, :] = s2n[None, :]

    @pl.when(i == G - 1)
    def _():
        mean = s1n / float(N)
        var = s2n / float(N) - mean * mean
        st_ref[0:1, :] = mean[None, :]
        st_ref[1:2, :] = var[None, :]


def _p4a(aggp, hs, counts, bias2d):
    return pl.pallas_call(
        _p4a_body,
        grid=(G,),
        in_specs=[
            pl.BlockSpec((NC, R, D), lambda i: (0, i, 0)),
            pl.BlockSpec((R, D), lambda i: (i, 0)),
            pl.BlockSpec((NT, R), lambda i: (0, i)),
            pl.BlockSpec((1, D), lambda i: (0, 0)),
        ],
        out_specs=[
            pl.BlockSpec((R, D), lambda i: (i, 0)),
            pl.BlockSpec((8, D), lambda i: (0, 0)),
        ],
        out_shape=[
            jax.ShapeDtypeStruct((NP, D), jnp.float32),
            jax.ShapeDtypeStruct((8, D), jnp.float32),
        ],
    )(aggp, hs, counts, bias2d)


def _p4b_body(pre_ref, st_ref, g_ref, b_ref, out_ref):
    mean = st_ref[0:1, :]
    var = st_ref[1:2, :]
    inv = lax.rsqrt(var + 1e-5)
    out_ref[...] = jnp.maximum(
        (pre_ref[...] - mean) * inv * g_ref[...] + b_ref[...], 0.0
    )


def _p4b(pre, stats, gamma2d, beta2d):
    return pl.pallas_call(
        _p4b_body,
        grid=(G,),
        in_specs=[
            pl.BlockSpec((R, D), lambda i: (i, 0)),
            pl.BlockSpec((8, D), lambda i: (0, 0)),
            pl.BlockSpec((1, D), lambda i: (0, 0)),
            pl.BlockSpec((1, D), lambda i: (0, 0)),
        ],
        out_specs=pl.BlockSpec((R, D), lambda i: (i, 0)),
        out_shape=jax.ShapeDtypeStruct((NP, D), jnp.float32),
    )(pre, stats, gamma2d, beta2d)


def kernel(x, edge_index, W, bias, gamma, beta):
    src = edge_index[0].astype(jnp.int32)
    dst = edge_index[1].astype(jnp.int32)
    pad_idx = jnp.full((EP - E,), N, jnp.int32)
    src_p = jnp.concatenate([src, pad_idx])
    dst_p = jnp.concatenate([dst, pad_idx])
    x_pad = jnp.pad(x, ((0, NP - N), (0, 0)))
    zeros2d = jnp.zeros((NP, D), jnp.float32)

    counts = _p1_hist(dst_p)
    hs = _p2(x_pad, W, counts)
    aggp = _p3_agg(hs, src_p, dst_p, zeros2d)
    pre, stats = _p4a(aggp, hs, counts, bias.reshape(1, D))
    out_pad = _p4b(pre, stats, gamma.reshape(1, D), beta.reshape(1, D))
    return out_pad[:N]


# speedup vs baseline: 20.9136x; 1.9934x over previous
import functools

import jax
import jax.numpy as jnp
from jax import lax
from jax.experimental import pallas as pl
from jax.experimental.pallas import tpu as pltpu
from jax.experimental.pallas import tpu_sc as plsc

N = 10000
D = 128
E = 320000

NP = 10240
EP = 327680
NC = 2
NS = 16
NT = NC * NS
EPT = EP // NT
CK = 128
NCHUNK = EPT // CK
RPT = NP // NS
R = 256
G = NP // R

_mesh = plsc.VectorSubcoreMesh(core_axis_name="c", subcore_axis_name="s")


@functools.partial(
    pl.kernel,
    mesh=_mesh,
    out_type=jax.ShapeDtypeStruct((NT, NP), jnp.float32),
    scratch_types=[
        pltpu.VMEM((EPT,), jnp.int32),
        pltpu.VMEM((NP,), jnp.float32),
    ],
    compiler_params=pltpu.CompilerParams(needs_layout_passes=False),
)
def _p1_hist(dst_hbm, counts_hbm, dst_v, hist):
    c = lax.axis_index("c")
    s = lax.axis_index("s")
    wid = c * NS + s

    def zbody(i, carry):
        hist[pl.ds(i * 16, 16)] = jnp.zeros((16,), jnp.float32)
        return carry

    lax.fori_loop(0, NP // 16, zbody, 0)

    pltpu.sync_copy(dst_hbm.at[pl.ds(wid * EPT, EPT)], dst_v)
    ones = jnp.full((16,), 1.0, jnp.float32)

    def body(g, carry):
        idx = dst_v[pl.ds(g * 16, 16)]
        plsc.addupdate_scatter(hist, [idx], ones)
        return carry

    lax.fori_loop(0, EPT // 16, body, 0)
    pltpu.sync_copy(hist, counts_hbm.at[wid])


@functools.partial(
    pl.kernel,
    mesh=_mesh,
    out_type=jax.ShapeDtypeStruct((NC, NP, D), jnp.float32),
    scratch_types=[
        pltpu.VMEM_SHARED((NP, D), jnp.float32),
        pltpu.VMEM((CK,), jnp.int32),
        pltpu.VMEM((CK,), jnp.int32),
        pltpu.VMEM((CK, D), jnp.float32),
        pltpu.SemaphoreType.DMA,
    ],
    compiler_params=pltpu.CompilerParams(needs_layout_passes=False),
)
def _p3_agg(hs_hbm, src_hbm, dst_hbm, zeros_hbm, out_hbm, agg, idx_s, idx_d, rows, sem):
    c = lax.axis_index("c")
    s = lax.axis_index("s")
    pltpu.sync_copy(zeros_hbm.at[pl.ds(s * RPT, RPT)], agg.at[pl.ds(s * RPT, RPT)])
    plsc.subcore_barrier()

    base0 = c * (EP // NC) + s * EPT

    def body(ch, carry):
        b = base0 + ch * CK
        pltpu.sync_copy(src_hbm.at[pl.ds(b, CK)], idx_s)
        pltpu.async_copy(hs_hbm.at[idx_s], rows, sem).wait()
        pltpu.sync_copy(dst_hbm.at[pl.ds(b, CK)], idx_d)
        pltpu.sync_copy(rows, agg.at[idx_d], add=True)
        return carry

    lax.fori_loop(0, NCHUNK, body, 0)
    plsc.subcore_barrier()
    pltpu.sync_copy(agg.at[pl.ds(s * RPT, RPT)], out_hbm.at[c, pl.ds(s * RPT, RPT)])


def _p2_body(x_ref, w_ref, cnt_ref, hs_ref):
    h = jnp.dot(x_ref[...], w_ref[...], preferred_element_type=jnp.float32)
    deg = jnp.sum(cnt_ref[...], axis=0) + 1.0
    dinv = lax.rsqrt(deg)
    hs_ref[...] = h * dinv[:, None]


def _p2(x_pad, W, counts):
    return pl.pallas_call(
        _p2_body,
        grid=(G,),
        in_specs=[
            pl.BlockSpec((R, D), lambda i: (i, 0)),
            pl.BlockSpec((D, D), lambda i: (0, 0)),
            pl.BlockSpec((NT, R), lambda i: (0, i)),
        ],
        out_specs=pl.BlockSpec((R, D), lambda i: (i, 0)),
        out_shape=jax.ShapeDtypeStruct((NP, D), jnp.float32),
    )(x_pad, W, counts)


def _p4a_body(agg_ref, hs_ref, cnt_ref, bias_ref, pre_ref, st_ref):
    i = pl.program_id(0)
    deg = jnp.sum(cnt_ref[...], axis=0) + 1.0
    dinv = lax.rsqrt(deg)
    tot = agg_ref[0] + agg_ref[1] + hs_ref[...]
    pre = tot * dinv[:, None] + bias_ref[...]
    pre_ref[...] = pre

    rows = i * R + lax.broadcasted_iota(jnp.int32, (R, 1), 0)
    m = (rows < N).astype(jnp.float32)
    pm = pre * m
    s1 = jnp.sum(pm, axis=0)
    s2 = jnp.sum(pm * pm, axis=0)
    prev = jnp.where(i == 0, 0.0, st_ref[...])
    s1n = prev[0] + s1
    s2n = prev[1] + s2
    st_ref[0:1, :] = s1n[None, :]
    st_ref[1:2, :] = s2n[None, :]

    @pl.when(i == G - 1)
    def _():
        mean = s1n / float(N)
        var = s2n / float(N) - mean * mean
        st_ref[0:1, :] = mean[None, :]
        st_ref[1:2, :] = var[None, :]


def _p4a(aggp, hs, counts, bias2d):
    return pl.pallas_call(
        _p4a_body,
        grid=(G,),
        in_specs=[
            pl.BlockSpec((NC, R, D), lambda i: (0, i, 0)),
            pl.BlockSpec((R, D), lambda i: (i, 0)),
            pl.BlockSpec((NT, R), lambda i: (0, i)),
            pl.BlockSpec((1, D), lambda i: (0, 0)),
        ],
        out_specs=[
            pl.BlockSpec((R, D), lambda i: (i, 0)),
            pl.BlockSpec((8, D), lambda i: (0, 0)),
        ],
        out_shape=[
            jax.ShapeDtypeStruct((NP, D), jnp.float32),
            jax.ShapeDtypeStruct((8, D), jnp.float32),
        ],
    )(aggp, hs, counts, bias2d)


def _p4b_body(pre_ref, st_ref, g_ref, b_ref, out_ref):
    mean = st_ref[0:1, :]
    var = st_ref[1:2, :]
    inv = lax.rsqrt(var + 1e-5)
    out_ref[...] = jnp.maximum(
        (pre_ref[...] - mean) * inv * g_ref[...] + b_ref[...], 0.0
    )


def _p4b(pre, stats, gamma2d, beta2d):
    return pl.pallas_call(
        _p4b_body,
        grid=(G,),
        in_specs=[
            pl.BlockSpec((R, D), lambda i: (i, 0)),
            pl.BlockSpec((8, D), lambda i: (0, 0)),
            pl.BlockSpec((1, D), lambda i: (0, 0)),
            pl.BlockSpec((1, D), lambda i: (0, 0)),
        ],
        out_specs=pl.BlockSpec((R, D), lambda i: (i, 0)),
        out_shape=jax.ShapeDtypeStruct((NP, D), jnp.float32),
    )(pre, stats, gamma2d, beta2d)


def kernel(x, edge_index, W, bias, gamma, beta):
    src = edge_index[0].astype(jnp.int32)
    dst = edge_index[1].astype(jnp.int32)
    pad_idx = N + jnp.arange(EP - E, dtype=jnp.int32) % (NP - N)
    src_p = jnp.concatenate([src, pad_idx])
    dst_p = jnp.concatenate([dst, pad_idx])
    x_pad = jnp.pad(x, ((0, NP - N), (0, 0)))
    zeros2d = jnp.zeros((NP, D), jnp.float32)

    counts = _p1_hist(dst_p)
    hs = _p2(x_pad, W, counts)
    aggp = _p3_agg(hs, src_p, dst_p, zeros2d)
    pre, stats = _p4a(aggp, hs, counts, bias.reshape(1, D))
    out_pad = _p4b(pre, stats, gamma.reshape(1, D), beta.reshape(1, D))
    return out_pad[:N]


# speedup vs baseline: 33.1558x; 1.5854x over previous
import functools

import jax
import jax.numpy as jnp
from jax import lax
from jax.experimental import pallas as pl
from jax.experimental.pallas import tpu as pltpu
from jax.experimental.pallas import tpu_sc as plsc

N = 10000
D = 128
E = 320000

NP = 10240
EP = 327680
NC = 2
NS = 16
NT = NC * NS
EPT = EP // NT
CK = 128
NCHUNK = EPT // CK
RPT = NP // NS
R = 256
G = NP // R

_mesh = plsc.VectorSubcoreMesh(core_axis_name="c", subcore_axis_name="s")


@functools.partial(
    pl.kernel,
    mesh=_mesh,
    out_type=jax.ShapeDtypeStruct((NT, NP), jnp.float32),
    scratch_types=[
        pltpu.VMEM((EPT,), jnp.int32),
        pltpu.VMEM((NP,), jnp.float32),
    ],
    compiler_params=pltpu.CompilerParams(needs_layout_passes=False),
)
def _p1_hist(dst_hbm, counts_hbm, dst_v, hist):
    c = lax.axis_index("c")
    s = lax.axis_index("s")
    wid = c * NS + s

    def zbody(i, carry):
        hist[pl.ds(i * 16, 16)] = jnp.zeros((16,), jnp.float32)
        return carry

    lax.fori_loop(0, NP // 16, zbody, 0)

    pltpu.sync_copy(dst_hbm.at[pl.ds(wid * EPT, EPT)], dst_v)
    ones = jnp.full((16,), 1.0, jnp.float32)

    def body(g, carry):
        idx = dst_v[pl.ds(g * 16, 16)]
        plsc.addupdate_scatter(hist, [idx], ones)
        return carry

    lax.fori_loop(0, EPT // 16, body, 0)
    pltpu.sync_copy(hist, counts_hbm.at[wid])


@functools.partial(
    pl.kernel,
    mesh=_mesh,
    out_type=jax.ShapeDtypeStruct((NC, NP, D), jnp.float32),
    scratch_types=[
        pltpu.VMEM_SHARED((NP, D), jnp.float32),
        pltpu.VMEM((NCHUNK // 2, CK), jnp.int32),
        pltpu.VMEM((NCHUNK // 2, CK), jnp.int32),
        pltpu.VMEM((CK, D), jnp.float32),
        pltpu.VMEM((CK, D), jnp.float32),
        pltpu.SemaphoreType.DMA,
        pltpu.SemaphoreType.DMA,
    ],
    compiler_params=pltpu.CompilerParams(needs_layout_passes=False),
)
def _p3_agg(
    hs_hbm, src_hbm, dst_hbm, zeros_hbm, out_hbm,
    agg, src_all, dst_all, rows_a, rows_b, sem_a, sem_b,
):
    c = lax.axis_index("c")
    s = lax.axis_index("s")
    pltpu.sync_copy(zeros_hbm.at[pl.ds(s * RPT, RPT)], agg.at[pl.ds(s * RPT, RPT)])
    plsc.subcore_barrier()

    HC = NCHUNK // 2
    cbase = c * (EP // NC // CK) + s * NCHUNK

    def gather(ch, rows, sem):
        return pltpu.make_async_copy(hs_hbm.at[src_all.at[ch]], rows, sem)

    for ph in range(2):
        pltpu.sync_copy(src_hbm.at[pl.ds(cbase + ph * HC, HC)], src_all)
        pltpu.sync_copy(dst_hbm.at[pl.ds(cbase + ph * HC, HC)], dst_all)
        gather(0, rows_a, sem_a).start()

        def body(i, carry):
            cha = 2 * i
            chb = 2 * i + 1
            gather(chb, rows_b, sem_b).start()
            gather(cha, rows_a, sem_a).wait()
            pltpu.sync_copy(rows_a, agg.at[dst_all.at[cha]], add=True)

            @pl.when(i < HC // 2 - 1)
            def _():
                gather(cha + 2, rows_a, sem_a).start()

            gather(chb, rows_b, sem_b).wait()
            pltpu.sync_copy(rows_b, agg.at[dst_all.at[chb]], add=True)
            return carry

        lax.fori_loop(0, HC // 2, body, 0)
    plsc.subcore_barrier()
    pltpu.sync_copy(agg.at[pl.ds(s * RPT, RPT)], out_hbm.at[c, pl.ds(s * RPT, RPT)])


def _p2_body(x_ref, w_ref, cnt_ref, hs_ref):
    h = jnp.dot(x_ref[...], w_ref[...], preferred_element_type=jnp.float32)
    deg = jnp.sum(cnt_ref[...], axis=0) + 1.0
    dinv = lax.rsqrt(deg)
    hs_ref[...] = h * dinv[:, None]


def _p2(x_pad, W, counts):
    return pl.pallas_call(
        _p2_body,
        grid=(G,),
        in_specs=[
            pl.BlockSpec((R, D), lambda i: (i, 0)),
            pl.BlockSpec((D, D), lambda i: (0, 0)),
            pl.BlockSpec((NT, R), lambda i: (0, i)),
        ],
        out_specs=pl.BlockSpec((R, D), lambda i: (i, 0)),
        out_shape=jax.ShapeDtypeStruct((NP, D), jnp.float32),
    )(x_pad, W, counts)


def _p4a_body(agg_ref, hs_ref, cnt_ref, bias_ref, pre_ref, st_ref):
    i = pl.program_id(0)
    deg = jnp.sum(cnt_ref[...], axis=0) + 1.0
    dinv = lax.rsqrt(deg)
    tot = agg_ref[0] + agg_ref[1] + hs_ref[...]
    pre = tot * dinv[:, None] + bias_ref[...]
    pre_ref[...] = pre

    rows = i * R + lax.broadcasted_iota(jnp.int32, (R, 1), 0)
    m = (rows < N).astype(jnp.float32)
    pm = pre * m
    s1 = jnp.sum(pm, axis=0)
    s2 = jnp.sum(pm * pm, axis=0)
    prev = jnp.where(i == 0, 0.0, st_ref[...])
    s1n = prev[0] + s1
    s2n = prev[1] + s2
    st_ref[0:1, :] = s1n[None, :]
    st_ref[1:2, :] = s2n[None, :]

    @pl.when(i == G - 1)
    def _():
        mean = s1n / float(N)
        var = s2n / float(N) - mean * mean
        st_ref[0:1, :] = mean[None, :]
        st_ref[1:2, :] = var[None, :]


def _p4a(aggp, hs, counts, bias2d):
    return pl.pallas_call(
        _p4a_body,
        grid=(G,),
        in_specs=[
            pl.BlockSpec((NC, R, D), lambda i: (0, i, 0)),
            pl.BlockSpec((R, D), lambda i: (i, 0)),
            pl.BlockSpec((NT, R), lambda i: (0, i)),
            pl.BlockSpec((1, D), lambda i: (0, 0)),
        ],
        out_specs=[
            pl.BlockSpec((R, D), lambda i: (i, 0)),
            pl.BlockSpec((8, D), lambda i: (0, 0)),
        ],
        out_shape=[
            jax.ShapeDtypeStruct((NP, D), jnp.float32),
            jax.ShapeDtypeStruct((8, D), jnp.float32),
        ],
    )(aggp, hs, counts, bias2d)


def _p4b_body(pre_ref, st_ref, g_ref, b_ref, out_ref):
    mean = st_ref[0:1, :]
    var = st_ref[1:2, :]
    inv = lax.rsqrt(var + 1e-5)
    out_ref[...] = jnp.maximum(
        (pre_ref[...] - mean) * inv * g_ref[...] + b_ref[...], 0.0
    )


def _p4b(pre, stats, gamma2d, beta2d):
    return pl.pallas_call(
        _p4b_body,
        grid=(G,),
        in_specs=[
            pl.BlockSpec((R, D), lambda i: (i, 0)),
            pl.BlockSpec((8, D), lambda i: (0, 0)),
            pl.BlockSpec((1, D), lambda i: (0, 0)),
            pl.BlockSpec((1, D), lambda i: (0, 0)),
        ],
        out_specs=pl.BlockSpec((R, D), lambda i: (i, 0)),
        out_shape=jax.ShapeDtypeStruct((NP, D), jnp.float32),
    )(pre, stats, gamma2d, beta2d)


def kernel(x, edge_index, W, bias, gamma, beta):
    src = edge_index[0].astype(jnp.int32)
    dst = edge_index[1].astype(jnp.int32)
    pad_idx = N + jnp.arange(EP - E, dtype=jnp.int32) % (NP - N)
    src_p = jnp.concatenate([src, pad_idx])
    dst_p = jnp.concatenate([dst, pad_idx])
    x_pad = jnp.pad(x, ((0, NP - N), (0, 0)))
    zeros2d = jnp.zeros((NP, D), jnp.float32)

    counts = _p1_hist(dst_p)
    hs = _p2(x_pad, W, counts)
    aggp = _p3_agg(hs, src_p.reshape(EP // CK, CK), dst_p.reshape(EP // CK, CK), zeros2d)
    pre, stats = _p4a(aggp, hs, counts, bias.reshape(1, D))
    out_pad = _p4b(pre, stats, gamma.reshape(1, D), beta.reshape(1, D))
    return out_pad[:N]


# speedup vs baseline: 34.7433x; 1.0479x over previous
import functools

import jax
import jax.numpy as jnp
from jax import lax
from jax.experimental import pallas as pl
from jax.experimental.pallas import tpu as pltpu
from jax.experimental.pallas import tpu_sc as plsc

N = 10000
D = 128
E = 320000

NP = 10240
EP = 327680
NC = 2
NS = 16
NT = NC * NS
EPT = EP // NT
CK = 128
NCHUNK = EPT // CK
RPT = NP // NS
R = 256
G = NP // R

_mesh = plsc.VectorSubcoreMesh(core_axis_name="c", subcore_axis_name="s")


@functools.partial(
    pl.kernel,
    mesh=_mesh,
    out_type=jax.ShapeDtypeStruct((NT, NP), jnp.float32),
    scratch_types=[
        pltpu.VMEM((EPT,), jnp.int32),
        pltpu.VMEM((NP,), jnp.float32),
    ],
    compiler_params=pltpu.CompilerParams(needs_layout_passes=False),
)
def _p1_hist(dst_hbm, counts_hbm, dst_v, hist):
    c = lax.axis_index("c")
    s = lax.axis_index("s")
    wid = c * NS + s

    def zbody(i, carry):
        hist[pl.ds(i * 16, 16)] = jnp.zeros((16,), jnp.float32)
        return carry

    lax.fori_loop(0, NP // 16, zbody, 0)

    pltpu.sync_copy(dst_hbm.at[pl.ds(wid * EPT, EPT)], dst_v)
    ones = jnp.full((16,), 1.0, jnp.float32)

    def body(g, carry):
        idx = dst_v[pl.ds(g * 16, 16)]
        plsc.addupdate_scatter(hist, [idx], ones)
        return carry

    lax.fori_loop(0, EPT // 16, body, 0)
    pltpu.sync_copy(hist, counts_hbm.at[wid])


@functools.partial(
    pl.kernel,
    mesh=_mesh,
    out_type=jax.ShapeDtypeStruct((NC, NP, D), jnp.float32),
    scratch_types=[
        pltpu.VMEM_SHARED((NP, D), jnp.float32),
        pltpu.VMEM((NCHUNK // 2, CK), jnp.int32),
        pltpu.VMEM((NCHUNK // 2, CK), jnp.int32),
        pltpu.VMEM((CK, D), jnp.float32),
        pltpu.VMEM((CK, D), jnp.float32),
        pltpu.SemaphoreType.DMA,
        pltpu.SemaphoreType.DMA,
    ],
    compiler_params=pltpu.CompilerParams(needs_layout_passes=False),
)
def _p3_agg(
    hs_hbm, src_hbm, dst_hbm, zeros_hbm, out_hbm,
    agg, src_all, dst_all, rows_a, rows_b, sem_a, sem_b,
):
    c = lax.axis_index("c")
    s = lax.axis_index("s")
    pltpu.sync_copy(zeros_hbm.at[pl.ds(s * RPT, RPT)], agg.at[pl.ds(s * RPT, RPT)])
    plsc.subcore_barrier()

    HC = NCHUNK // 2
    cbase = c * (EP // NC // CK) + s * NCHUNK

    def gather(ch, rows, sem):
        return pltpu.make_async_copy(hs_hbm.at[src_all.at[ch]], rows, sem)

    for ph in range(2):
        pltpu.sync_copy(src_hbm.at[pl.ds(cbase + ph * HC, HC)], src_all)
        pltpu.sync_copy(dst_hbm.at[pl.ds(cbase + ph * HC, HC)], dst_all)
        gather(0, rows_a, sem_a).start()

        def body(i, carry):
            cha = 2 * i
            chb = 2 * i + 1
            gather(chb, rows_b, sem_b).start()
            gather(cha, rows_a, sem_a).wait()
            pltpu.sync_copy(rows_a, agg.at[dst_all.at[cha]], add=True)

            @pl.when(i < HC // 2 - 1)
            def _():
                gather(cha + 2, rows_a, sem_a).start()

            gather(chb, rows_b, sem_b).wait()
            pltpu.sync_copy(rows_b, agg.at[dst_all.at[chb]], add=True)
            return carry

        lax.fori_loop(0, HC // 2, body, 0)
    plsc.subcore_barrier()
    pltpu.sync_copy(agg.at[pl.ds(s * RPT, RPT)], out_hbm.at[c, pl.ds(s * RPT, RPT)])


def _p2_body(x_ref, w_ref, cnt_ref, hs_ref):
    h = jnp.dot(x_ref[...], w_ref[...], preferred_element_type=jnp.float32)
    deg = jnp.sum(cnt_ref[...], axis=0) + 1.0
    dinv = lax.rsqrt(deg)
    hs_ref[...] = h * dinv[:, None]


def _p2(x_pad, W, counts):
    return pl.pallas_call(
        _p2_body,
        grid=(G,),
        in_specs=[
            pl.BlockSpec((R, D), lambda i: (i, 0)),
            pl.BlockSpec((D, D), lambda i: (0, 0)),
            pl.BlockSpec((NT, R), lambda i: (0, i)),
        ],
        out_specs=pl.BlockSpec((R, D), lambda i: (i, 0)),
        out_shape=jax.ShapeDtypeStruct((NP, D), jnp.float32),
    )(x_pad, W, counts)


def _p4_body(agg_ref, hs_ref, cnt_ref, bias_ref, gamma_ref, beta_ref,
             out_ref, pre_scr, st_scr):
    i = pl.program_id(0)

    @pl.when(i < G)
    def _():
        deg = jnp.sum(cnt_ref[...], axis=0) + 1.0
        dinv = lax.rsqrt(deg)
        tot = agg_ref[0] + agg_ref[1] + hs_ref[...]
        pre = tot * dinv[:, None] + bias_ref[...]
        pre_scr[pl.ds(i * R, R), :] = pre
        rows = i * R + lax.broadcasted_iota(jnp.int32, (R, 1), 0)
        m = (rows < N).astype(jnp.float32)
        pm = pre * m
        s1 = jnp.sum(pm, axis=0)
        s2 = jnp.sum(pm * pm, axis=0)
        prev = jnp.where(i == 0, 0.0, st_scr[...])
        st_scr[0:1, :] = (prev[0] + s1)[None, :]
        st_scr[1:2, :] = (prev[1] + s2)[None, :]

    @pl.when(i >= G)
    def _():
        j = i - G
        mean = st_scr[0:1, :] / float(N)
        var = st_scr[1:2, :] / float(N) - mean * mean
        inv = lax.rsqrt(var + 1e-5)
        pre = pre_scr[pl.ds(j * R, R), :]
        out_ref[...] = jnp.maximum(
            (pre - mean) * inv * gamma_ref[...] + beta_ref[...], 0.0
        )


def _p4(aggp, hs, counts, bias2d, gamma2d, beta2d):
    def rowmap(i):
        return jnp.minimum(i, G - 1)

    return pl.pallas_call(
        _p4_body,
        grid=(2 * G,),
        in_specs=[
            pl.BlockSpec((NC, R, D), lambda i: (0, rowmap(i), 0)),
            pl.BlockSpec((R, D), lambda i: (rowmap(i), 0)),
            pl.BlockSpec((NT, R), lambda i: (0, rowmap(i))),
            pl.BlockSpec((1, D), lambda i: (0, 0)),
            pl.BlockSpec((1, D), lambda i: (0, 0)),
            pl.BlockSpec((1, D), lambda i: (0, 0)),
        ],
        out_specs=pl.BlockSpec(
            (R, D), lambda i: (jnp.where(i < G, 0, i - G), 0)
        ),
        out_shape=jax.ShapeDtypeStruct((NP, D), jnp.float32),
        scratch_shapes=[
            pltpu.VMEM((NP, D), jnp.float32),
            pltpu.VMEM((8, D), jnp.float32),
        ],
    )(aggp, hs, counts, bias2d, gamma2d, beta2d)


def kernel(x, edge_index, W, bias, gamma, beta):
    src = edge_index[0].astype(jnp.int32)
    dst = edge_index[1].astype(jnp.int32)
    pad_idx = N + jnp.arange(EP - E, dtype=jnp.int32) % (NP - N)
    src_p = jnp.concatenate([src, pad_idx])
    dst_p = jnp.concatenate([dst, pad_idx])
    x_pad = jnp.pad(x, ((0, NP - N), (0, 0)))
    zeros2d = jnp.zeros((NP, D), jnp.float32)

    counts = _p1_hist(dst_p)
    hs = _p2(x_pad, W, counts)
    aggp = _p3_agg(hs, src_p.reshape(EP // CK, CK), dst_p.reshape(EP // CK, CK), zeros2d)
    out_pad = _p4(aggp, hs, counts, bias.reshape(1, D),
                  gamma.reshape(1, D), beta.reshape(1, D))
    return out_pad[:N]


# speedup vs baseline: 36.2743x; 1.0441x over previous
import functools

import jax
import jax.numpy as jnp
from jax import lax
from jax.experimental import pallas as pl
from jax.experimental.pallas import tpu as pltpu
from jax.experimental.pallas import tpu_sc as plsc

N = 10000
D = 128
E = 320000

NC = 2
NS = 16
NT = NC * NS
EPT = E // NT
CK = 80
NCHUNK = EPT // CK
R = 400
G = N // R

_mesh = plsc.VectorSubcoreMesh(core_axis_name="c", subcore_axis_name="s")


@functools.partial(
    pl.kernel,
    mesh=_mesh,
    out_type=jax.ShapeDtypeStruct((NT, N), jnp.float32),
    scratch_types=[
        pltpu.VMEM((EPT,), jnp.int32),
        pltpu.VMEM((N,), jnp.float32),
    ],
    compiler_params=pltpu.CompilerParams(needs_layout_passes=False),
)
def _p1_hist(dst_hbm, counts_hbm, dst_v, hist):
    c = lax.axis_index("c")
    s = lax.axis_index("s")
    wid = c * NS + s

    def zbody(i, carry):
        hist[pl.ds(i * 16, 16)] = jnp.zeros((16,), jnp.float32)
        return carry

    lax.fori_loop(0, N // 16, zbody, 0)

    pltpu.sync_copy(dst_hbm.at[pl.ds(wid * EPT, EPT)], dst_v)
    ones = jnp.full((16,), 1.0, jnp.float32)

    def body(g, carry):
        idx = dst_v[pl.ds(g * 16, 16)]
        plsc.addupdate_scatter(hist, [idx], ones)
        return carry

    lax.fori_loop(0, EPT // 16, body, 0)
    pltpu.sync_copy(hist, counts_hbm.at[wid])


@functools.partial(
    pl.kernel,
    mesh=_mesh,
    out_type=jax.ShapeDtypeStruct((NC, N, D), jnp.float32),
    scratch_types=[
        pltpu.VMEM_SHARED((N, D), jnp.float32),
        pltpu.VMEM((EPT,), jnp.int32),
        pltpu.VMEM((NCHUNK, CK), jnp.int32),
        pltpu.VMEM((CK, D), jnp.float32),
        pltpu.VMEM((CK, D), jnp.float32),
        pltpu.SemaphoreType.DMA,
        pltpu.SemaphoreType.DMA,
    ],
    compiler_params=pltpu.CompilerParams(needs_layout_passes=False),
)
def _p3_agg(
    hs_hbm, src_hbm, dst_hbm, zeros_hbm, out_hbm,
    agg, src_all, dst_all, rows_a, rows_b, sem_a, sem_b,
):
    c = lax.axis_index("c")
    s = lax.axis_index("s")
    wid = c * NS + s
    pltpu.sync_copy(src_hbm.at[pl.ds(wid * EPT, EPT)], src_all)
    pltpu.sync_copy(dst_hbm.at[wid], dst_all)
    pltpu.sync_copy(zeros_hbm.at[pl.ds(s * 624, 624)], agg.at[pl.ds(s * 624, 624)])

    @pl.when(s == NS - 1)
    def _():
        pltpu.sync_copy(zeros_hbm.at[pl.ds(9984, 16)], agg.at[pl.ds(9984, 16)])

    plsc.subcore_barrier()

    def gather(ch, rows, sem):
        return pltpu.make_async_copy(
            hs_hbm.at[src_all.at[pl.ds(ch * CK, CK)]], rows, sem
        )

    def scatter(ch, rows):
        pltpu.sync_copy(rows, agg.at[dst_all.at[ch]], add=True)

    gather(0, rows_a, sem_a).start()

    def body(i, carry):
        cha = 2 * i
        chb = 2 * i + 1
        gather(chb, rows_b, sem_b).start()
        gather(cha, rows_a, sem_a).wait()
        scatter(cha, rows_a)
        gather(cha + 2, rows_a, sem_a).start()
        gather(chb, rows_b, sem_b).wait()
        scatter(chb, rows_b)
        return carry

    lax.fori_loop(0, NCHUNK // 2, body, 0)
    gather(NCHUNK - 1, rows_a, sem_a).wait()
    scatter(NCHUNK - 1, rows_a)

    plsc.subcore_barrier()
    pltpu.sync_copy(agg.at[pl.ds(s * 624, 624)], out_hbm.at[c, pl.ds(s * 624, 624)])

    @pl.when(s == NS - 1)
    def _():
        pltpu.sync_copy(agg.at[pl.ds(9984, 16)], out_hbm.at[c, pl.ds(9984, 16)])


def _p2_body(x_ref, w_ref, cnt_ref, hs_ref):
    h = jnp.dot(x_ref[...], w_ref[...], preferred_element_type=jnp.float32)
    deg = jnp.sum(cnt_ref[0], axis=-1) + 1.0
    dinv = lax.rsqrt(deg)
    hs_ref[...] = h * dinv[:, None]


def _p2(x, W, counts):
    return pl.pallas_call(
        _p2_body,
        grid=(G,),
        in_specs=[
            pl.BlockSpec((R, D), lambda i: (i, 0)),
            pl.BlockSpec((D, D), lambda i: (0, 0)),
            pl.BlockSpec((1, R, NT), lambda i: (i, 0, 0)),
        ],
        out_specs=pl.BlockSpec((R, D), lambda i: (i, 0)),
        out_shape=jax.ShapeDtypeStruct((N, D), jnp.float32),
    )(x, W, counts)


def _p4_body(agg_ref, hs_ref, cnt_ref, bias_ref, gamma_ref, beta_ref,
             out_ref, pre_scr, st_scr):
    i = pl.program_id(0)

    @pl.when(i < G)
    def _():
        deg = jnp.sum(cnt_ref[0], axis=-1) + 1.0
        dinv = lax.rsqrt(deg)
        tot = agg_ref[0] + agg_ref[1] + hs_ref[...]
        pre = tot * dinv[:, None] + bias_ref[...]
        pre_scr[pl.ds(pl.multiple_of(i * R, 8), R), :] = pre
        s1 = jnp.sum(pre, axis=0)
        s2 = jnp.sum(pre * pre, axis=0)
        prev = jnp.where(i == 0, 0.0, st_scr[...])
        st_scr[0:1, :] = (prev[0] + s1)[None, :]
        st_scr[1:2, :] = (prev[1] + s2)[None, :]

    @pl.when(i >= G)
    def _():
        j = i - G
        mean = st_scr[0:1, :] / float(N)
        var = st_scr[1:2, :] / float(N) - mean * mean
        inv = lax.rsqrt(var + 1e-5)
        pre = pre_scr[pl.ds(pl.multiple_of(j * R, 8), R), :]
        out_ref[...] = jnp.maximum(
            (pre - mean) * inv * gamma_ref[...] + beta_ref[...], 0.0
        )


def _p4(aggp, hs, counts, bias2d, gamma2d, beta2d):
    def rowmap(i):
        return jnp.minimum(i, G - 1)

    return pl.pallas_call(
        _p4_body,
        grid=(2 * G,),
        in_specs=[
            pl.BlockSpec((NC, R, D), lambda i: (0, rowmap(i), 0)),
            pl.BlockSpec((R, D), lambda i: (rowmap(i), 0)),
            pl.BlockSpec((1, R, NT), lambda i: (rowmap(i), 0, 0)),
            pl.BlockSpec((1, D), lambda i: (0, 0)),
            pl.BlockSpec((1, D), lambda i: (0, 0)),
            pl.BlockSpec((1, D), lambda i: (0, 0)),
        ],
        out_specs=pl.BlockSpec(
            (R, D), lambda i: (jnp.where(i < G, 0, i - G), 0)
        ),
        out_shape=jax.ShapeDtypeStruct((N, D), jnp.float32),
        scratch_shapes=[
            pltpu.VMEM((N, D), jnp.float32),
            pltpu.VMEM((8, D), jnp.float32),
        ],
    )(aggp, hs, counts, bias2d, gamma2d, beta2d)


def kernel(x, edge_index, W, bias, gamma, beta):
    src = edge_index[0].astype(jnp.int32)
    dst = edge_index[1].astype(jnp.int32)
    zeros2d = jnp.zeros((N, D), jnp.float32)

    counts = _p1_hist(dst)
    counts_t = counts.T.reshape(G, R, NT)
    hs = _p2(x, W, counts_t)
    aggp = _p3_agg(hs, src, dst.reshape(NT, NCHUNK, CK), zeros2d)
    return _p4(aggp, hs, counts_t, bias.reshape(1, D),
               gamma.reshape(1, D), beta.reshape(1, D))


# speedup vs baseline: 38.4221x; 1.0592x over previous
import functools

import jax
import jax.numpy as jnp
from jax import lax
from jax.experimental import pallas as pl
from jax.experimental.pallas import tpu as pltpu
from jax.experimental.pallas import tpu_sc as plsc

N = 10000
D = 128
E = 320000

NC = 2
NS = 16
NT = NC * NS
EPT = E // NT
NFAKE = 112
EPTP = EPT + NFAKE
CK = 128
NCHUNK = EPTP // CK
HCH = 40
NSELF = NT * NFAKE
R = 400
G = N // R

_mesh = plsc.VectorSubcoreMesh(core_axis_name="c", subcore_axis_name="s")


@functools.partial(
    pl.kernel,
    mesh=_mesh,
    out_type=jax.ShapeDtypeStruct((NT, N), jnp.float32),
    scratch_types=[
        pltpu.VMEM((EPT,), jnp.int32),
        pltpu.VMEM((N,), jnp.float32),
    ],
    compiler_params=pltpu.CompilerParams(needs_layout_passes=False),
)
def _p1_hist(dst_hbm, counts_hbm, dst_v, hist):
    c = lax.axis_index("c")
    s = lax.axis_index("s")
    wid = c * NS + s

    def zbody(i, carry):
        hist[pl.ds(i * 16, 16)] = jnp.zeros((16,), jnp.float32)
        return carry

    lax.fori_loop(0, N // 16, zbody, 0)

    pltpu.sync_copy(dst_hbm.at[pl.ds(wid * EPT, EPT)], dst_v)
    ones = jnp.full((16,), 1.0, jnp.float32)

    def body(g, carry):
        idx = dst_v[pl.ds(g * 16, 16)]
        plsc.addupdate_scatter(hist, [idx], ones)
        return carry

    lax.fori_loop(0, EPT // 16, body, 0)
    pltpu.sync_copy(hist, counts_hbm.at[wid])


@functools.partial(
    pl.kernel,
    mesh=_mesh,
    out_type=jax.ShapeDtypeStruct((NC, N, D), jnp.float32),
    scratch_types=[
        pltpu.VMEM_SHARED((N, D), jnp.float32),
        pltpu.VMEM((EPTP,), jnp.int32),
        pltpu.VMEM((HCH, CK), jnp.int32),
        pltpu.VMEM((CK, D), jnp.float32),
        pltpu.VMEM((CK, D), jnp.float32),
        pltpu.SemaphoreType.DMA,
        pltpu.SemaphoreType.DMA,
    ],
    compiler_params=pltpu.CompilerParams(needs_layout_passes=False),
)
def _p3_agg(
    hs_hbm, src_hbm, dst_hbm, zeros_hbm, out_hbm,
    agg, src_all, dst_all, rows_a, rows_b, sem_a, sem_b,
):
    c = lax.axis_index("c")
    s = lax.axis_index("s")
    wid = c * NS + s
    pltpu.sync_copy(src_hbm.at[pl.ds(wid * EPTP, EPTP)], src_all)
    pltpu.sync_copy(dst_hbm.at[wid, pl.ds(0, HCH)], dst_all)
    pltpu.sync_copy(zeros_hbm.at[pl.ds(s * 624, 624)], agg.at[pl.ds(s * 624, 624)])

    @pl.when(s == NS - 1)
    def _():
        pltpu.sync_copy(zeros_hbm.at[pl.ds(9984, 16)], agg.at[pl.ds(9984, 16)])

    plsc.subcore_barrier()

    def gather(ch, rows, sem):
        return pltpu.make_async_copy(
            hs_hbm.at[src_all.at[pl.ds(ch * CK, CK)]], rows, sem
        )

    def scatter(ch, rows):
        row = jnp.where(ch < HCH, ch, ch - HCH)
        pltpu.sync_copy(rows, agg.at[dst_all.at[row]], add=True)

    gather(0, rows_a, sem_a).start()

    def body(i, carry):
        cha = 2 * i
        chb = 2 * i + 1
        gather(chb, rows_b, sem_b).start()

        @pl.when(i == HCH // 2)
        def _():
            pltpu.sync_copy(
                dst_hbm.at[wid, pl.ds(HCH, NCHUNK - HCH)],
                dst_all.at[pl.ds(0, NCHUNK - HCH)],
            )

        gather(cha, rows_a, sem_a).wait()
        scatter(cha, rows_a)
        gather(cha + 2, rows_a, sem_a).start()
        gather(chb, rows_b, sem_b).wait()
        scatter(chb, rows_b)
        return carry

    lax.fori_loop(0, NCHUNK // 2, body, 0)
    gather(NCHUNK - 1, rows_a, sem_a).wait()
    scatter(NCHUNK - 1, rows_a)

    plsc.subcore_barrier()
    pltpu.sync_copy(agg.at[pl.ds(s * 624, 624)], out_hbm.at[c, pl.ds(s * 624, 624)])

    @pl.when(s == NS - 1)
    def _():
        pltpu.sync_copy(agg.at[pl.ds(9984, 16)], out_hbm.at[c, pl.ds(9984, 16)])


def _p2_body(x_ref, w_ref, cnt_ref, hs_ref):
    h = jnp.dot(x_ref[...], w_ref[...], preferred_element_type=jnp.float32)
    deg = jnp.sum(cnt_ref[0], axis=-1) + 1.0
    dinv = lax.rsqrt(deg)
    hs_ref[...] = h * dinv[:, None]


def _p2(x, W, counts):
    return pl.pallas_call(
        _p2_body,
        grid=(G,),
        in_specs=[
            pl.BlockSpec((R, D), lambda i: (i, 0)),
            pl.BlockSpec((D, D), lambda i: (0, 0)),
            pl.BlockSpec((1, R, NT), lambda i: (i, 0, 0)),
        ],
        out_specs=pl.BlockSpec((R, D), lambda i: (i, 0)),
        out_shape=jax.ShapeDtypeStruct((N, D), jnp.float32),
    )(x, W, counts)


def _p4_body(agg_ref, hs_ref, cnt_ref, bias_ref, gamma_ref, beta_ref,
             out_ref, pre_scr, st_scr):
    i = pl.program_id(0)

    @pl.when(i < G)
    def _():
        deg = jnp.sum(cnt_ref[0], axis=-1) + 1.0
        dinv = lax.rsqrt(deg)
        rows = i * R + lax.broadcasted_iota(jnp.int32, (R, 1), 0)
        selfmask = (rows >= NSELF).astype(jnp.float32)
        tot = agg_ref[0] + agg_ref[1] + hs_ref[...] * selfmask
        pre = tot * dinv[:, None] + bias_ref[...]
        pre_scr[pl.ds(pl.multiple_of(i * R, 8), R), :] = pre
        s1 = jnp.sum(pre, axis=0)
        s2 = jnp.sum(pre * pre, axis=0)
        prev = jnp.where(i == 0, 0.0, st_scr[...])
        st_scr[0:1, :] = (prev[0] + s1)[None, :]
        st_scr[1:2, :] = (prev[1] + s2)[None, :]

    @pl.when(i >= G)
    def _():
        j = i - G
        mean = st_scr[0:1, :] / float(N)
        var = st_scr[1:2, :] / float(N) - mean * mean
        inv = lax.rsqrt(var + 1e-5)
        pre = pre_scr[pl.ds(pl.multiple_of(j * R, 8), R), :]
        out_ref[...] = jnp.maximum(
            (pre - mean) * inv * gamma_ref[...] + beta_ref[...], 0.0
        )


def _p4(aggp, hs, counts, bias2d, gamma2d, beta2d):
    def rowmap(i):
        return jnp.minimum(i, G - 1)

    return pl.pallas_call(
        _p4_body,
        grid=(2 * G,),
        in_specs=[
            pl.BlockSpec((NC, R, D), lambda i: (0, rowmap(i), 0)),
            pl.BlockSpec((R, D), lambda i: (rowmap(i), 0)),
            pl.BlockSpec((1, R, NT), lambda i: (rowmap(i), 0, 0)),
            pl.BlockSpec((1, D), lambda i: (0, 0)),
            pl.BlockSpec((1, D), lambda i: (0, 0)),
            pl.BlockSpec((1, D), lambda i: (0, 0)),
        ],
        out_specs=pl.BlockSpec(
            (R, D), lambda i: (jnp.where(i < G, 0, i - G), 0)
        ),
        out_shape=jax.ShapeDtypeStruct((N, D), jnp.float32),
        scratch_shapes=[
            pltpu.VMEM((N, D), jnp.float32),
            pltpu.VMEM((8, D), jnp.float32),
        ],
    )(aggp, hs, counts, bias2d, gamma2d, beta2d)


def kernel(x, edge_index, W, bias, gamma, beta):
    src = edge_index[0].astype(jnp.int32)
    dst = edge_index[1].astype(jnp.int32)
    zeros2d = jnp.zeros((N, D), jnp.float32)

    counts = _p1_hist(dst)
    counts_t = counts.T.reshape(G, R, NT)
    hs = _p2(x, W, counts_t)
    fake = (jnp.arange(NT, dtype=jnp.int32)[:, None] * NFAKE
            + jnp.arange(NFAKE, dtype=jnp.int32)[None, :])
    src_p = jnp.concatenate([src.reshape(NT, EPT), fake], axis=1).reshape(-1)
    dst_p = jnp.concatenate([dst.reshape(NT, EPT), fake], axis=1)
    aggp = _p3_agg(hs, src_p, dst_p.reshape(NT, NCHUNK, CK), zeros2d)
    return _p4(aggp, hs, counts_t, bias.reshape(1, D),
               gamma.reshape(1, D), beta.reshape(1, D))


# speedup vs baseline: 43.8846x; 1.1422x over previous
import functools

import jax
import jax.numpy as jnp
from jax import lax
from jax.experimental import pallas as pl
from jax.experimental.pallas import tpu as pltpu
from jax.experimental.pallas import tpu_sc as plsc

N = 10000
D = 128
E = 320000

NC = 2
NS = 16
NT = NC * NS
EPT = E // NT
NFAKE = 112
EPTP = EPT + NFAKE
CK = 128
NCHUNK = EPTP // CK
HCH = 40
NSELF = NT * NFAKE
R = 2000
G = N // R

_mesh = plsc.VectorSubcoreMesh(core_axis_name="c", subcore_axis_name="s")


@functools.partial(
    pl.kernel,
    mesh=_mesh,
    out_type=jax.ShapeDtypeStruct((NT, N), jnp.float32),
    scratch_types=[
        pltpu.VMEM((EPT,), jnp.int32),
        pltpu.VMEM((N,), jnp.float32),
    ],
    compiler_params=pltpu.CompilerParams(needs_layout_passes=False),
)
def _p1_hist(dst_hbm, counts_hbm, dst_v, hist):
    c = lax.axis_index("c")
    s = lax.axis_index("s")
    wid = c * NS + s

    def zbody(i, carry):
        hist[pl.ds(i * 16, 16)] = jnp.zeros((16,), jnp.float32)
        return carry

    lax.fori_loop(0, N // 16, zbody, 0)

    pltpu.sync_copy(dst_hbm.at[pl.ds(wid * EPT, EPT)], dst_v)
    ones = jnp.full((16,), 1.0, jnp.float32)

    def body(g, carry):
        idx = dst_v[pl.ds(g * 16, 16)]
        plsc.addupdate_scatter(hist, [idx], ones)
        return carry

    lax.fori_loop(0, EPT // 16, body, 0)
    pltpu.sync_copy(hist, counts_hbm.at[wid])


@functools.partial(
    pl.kernel,
    mesh=_mesh,
    out_type=jax.ShapeDtypeStruct((NC, N, D), jnp.float32),
    scratch_types=[
        pltpu.VMEM_SHARED((N, D), jnp.float32),
        pltpu.VMEM((EPTP,), jnp.int32),
        pltpu.VMEM((HCH, CK), jnp.int32),
        pltpu.VMEM((CK, D), jnp.float32),
        pltpu.VMEM((CK, D), jnp.float32),
        pltpu.SemaphoreType.DMA,
        pltpu.SemaphoreType.DMA,
    ],
    compiler_params=pltpu.CompilerParams(needs_layout_passes=False),
)
def _p3_agg(
    hs_hbm, src_hbm, dst_hbm, zeros_hbm, out_hbm,
    agg, src_all, dst_all, rows_a, rows_b, sem_a, sem_b,
):
    c = lax.axis_index("c")
    s = lax.axis_index("s")
    wid = c * NS + s
    pltpu.sync_copy(src_hbm.at[pl.ds(wid * EPTP, EPTP)], src_all)
    pltpu.sync_copy(dst_hbm.at[wid, pl.ds(0, HCH)], dst_all)
    pltpu.sync_copy(zeros_hbm.at[pl.ds(s * 624, 624)], agg.at[pl.ds(s * 624, 624)])

    @pl.when(s == NS - 1)
    def _():
        pltpu.sync_copy(zeros_hbm.at[pl.ds(9984, 16)], agg.at[pl.ds(9984, 16)])

    plsc.subcore_barrier()

    def gather(ch, rows, sem):
        return pltpu.make_async_copy(
            hs_hbm.at[src_all.at[pl.ds(ch * CK, CK)]], rows, sem
        )

    def scatter(ch, rows):
        row = jnp.where(ch < HCH, ch, ch - HCH)
        pltpu.sync_copy(rows, agg.at[dst_all.at[row]], add=True)

    gather(0, rows_a, sem_a).start()

    def body(i, carry):
        cha = 2 * i
        chb = 2 * i + 1
        gather(chb, rows_b, sem_b).start()

        @pl.when(i == HCH // 2)
        def _():
            pltpu.sync_copy(
                dst_hbm.at[wid, pl.ds(HCH, NCHUNK - HCH)],
                dst_all.at[pl.ds(0, NCHUNK - HCH)],
            )

        gather(cha, rows_a, sem_a).wait()
        scatter(cha, rows_a)
        gather(cha + 2, rows_a, sem_a).start()
        gather(chb, rows_b, sem_b).wait()
        scatter(chb, rows_b)
        return carry

    lax.fori_loop(0, NCHUNK // 2, body, 0)
    gather(NCHUNK - 1, rows_a, sem_a).wait()
    scatter(NCHUNK - 1, rows_a)

    plsc.subcore_barrier()
    pltpu.sync_copy(agg.at[pl.ds(s * 624, 624)], out_hbm.at[c, pl.ds(s * 624, 624)])

    @pl.when(s == NS - 1)
    def _():
        pltpu.sync_copy(agg.at[pl.ds(9984, 16)], out_hbm.at[c, pl.ds(9984, 16)])


def _p2_body(x_ref, w_ref, cnt_ref, hs_ref):
    h = jnp.dot(x_ref[...], w_ref[...], preferred_element_type=jnp.float32)
    deg = jnp.sum(cnt_ref[0], axis=-1) + 1.0
    dinv = lax.rsqrt(deg)
    hs_ref[...] = h * dinv[:, None]


def _p2(x, W, counts):
    return pl.pallas_call(
        _p2_body,
        grid=(G,),
        in_specs=[
            pl.BlockSpec((R, D), lambda i: (i, 0)),
            pl.BlockSpec((D, D), lambda i: (0, 0)),
            pl.BlockSpec((1, R, NT), lambda i: (i, 0, 0)),
        ],
        out_specs=pl.BlockSpec((R, D), lambda i: (i, 0)),
        out_shape=jax.ShapeDtypeStruct((N, D), jnp.float32),
    )(x, W, counts)


def _p4_body(agg_ref, hs_ref, cnt_ref, bias_ref, gamma_ref, beta_ref,
             out_ref, pre_scr, st_scr):
    i = pl.program_id(0)

    @pl.when(i < G)
    def _():
        deg = jnp.sum(cnt_ref[0], axis=-1) + 1.0
        dinv = lax.rsqrt(deg)
        rows = i * R + lax.broadcasted_iota(jnp.int32, (R, 1), 0)
        selfmask = (rows >= NSELF).astype(jnp.float32)
        tot = agg_ref[0] + agg_ref[1] + hs_ref[...] * selfmask
        pre = tot * dinv[:, None] + bias_ref[...]
        pre_scr[pl.ds(pl.multiple_of(i * R, 8), R), :] = pre
        s1 = jnp.sum(pre, axis=0)
        s2 = jnp.sum(pre * pre, axis=0)
        prev = jnp.where(i == 0, 0.0, st_scr[...])
        st_scr[0:1, :] = (prev[0] + s1)[None, :]
        st_scr[1:2, :] = (prev[1] + s2)[None, :]

    @pl.when(i >= G)
    def _():
        j = i - G
        mean = st_scr[0:1, :] / float(N)
        var = st_scr[1:2, :] / float(N) - mean * mean
        inv = lax.rsqrt(var + 1e-5)
        pre = pre_scr[pl.ds(pl.multiple_of(j * R, 8), R), :]
        out_ref[...] = jnp.maximum(
            (pre - mean) * inv * gamma_ref[...] + beta_ref[...], 0.0
        )


def _p4(aggp, hs, counts, bias2d, gamma2d, beta2d):
    def rowmap(i):
        return jnp.minimum(i, G - 1)

    return pl.pallas_call(
        _p4_body,
        grid=(2 * G,),
        in_specs=[
            pl.BlockSpec((NC, R, D), lambda i: (0, rowmap(i), 0)),
            pl.BlockSpec((R, D), lambda i: (rowmap(i), 0)),
            pl.BlockSpec((1, R, NT), lambda i: (rowmap(i), 0, 0)),
            pl.BlockSpec((1, D), lambda i: (0, 0)),
            pl.BlockSpec((1, D), lambda i: (0, 0)),
            pl.BlockSpec((1, D), lambda i: (0, 0)),
        ],
        out_specs=pl.BlockSpec(
            (R, D), lambda i: (jnp.where(i < G, 0, i - G), 0)
        ),
        out_shape=jax.ShapeDtypeStruct((N, D), jnp.float32),
        scratch_shapes=[
            pltpu.VMEM((N, D), jnp.float32),
            pltpu.VMEM((8, D), jnp.float32),
        ],
    )(aggp, hs, counts, bias2d, gamma2d, beta2d)


def kernel(x, edge_index, W, bias, gamma, beta):
    src = edge_index[0].astype(jnp.int32)
    dst = edge_index[1].astype(jnp.int32)
    zeros2d = jnp.zeros((N, D), jnp.float32)

    counts = _p1_hist(dst)
    counts_t = counts.T.reshape(G, R, NT)
    hs = _p2(x, W, counts_t)
    fake = (jnp.arange(NT, dtype=jnp.int32)[:, None] * NFAKE
            + jnp.arange(NFAKE, dtype=jnp.int32)[None, :])
    src_p = jnp.concatenate([src.reshape(NT, EPT), fake], axis=1).reshape(-1)
    dst_p = jnp.concatenate([dst.reshape(NT, EPT), fake], axis=1)
    aggp = _p3_agg(hs, src_p, dst_p.reshape(NT, NCHUNK, CK), zeros2d)
    return _p4(aggp, hs, counts_t, bias.reshape(1, D),
               gamma.reshape(1, D), beta.reshape(1, D))
